# baseline (speedup 1.0000x reference)
import functools

import jax
import jax.numpy as jnp
from jax import lax
from jax.experimental import pallas as pl
from jax.experimental.pallas import tpu as pltpu
from jax.experimental.pallas import tpu_sc as plsc

B, S, V = 32, 32, 100000
N = B * S
L = 16
NTILES = 16
PER_TILE = N // NTILES
CHUNKS = PER_TILE // L


def _sc_body(pred_hbm, tgt_hbm, rew_hbm, out_hbm,
             tgt_v, rew_v, fidx_v, vals_v, accv_v, red_v, out_v,
             shared, sem):
    c = lax.axis_index("c")
    s = lax.axis_index("s")

    @pl.when(c == 0)
    def _gather_and_partial():
        base = pl.multiple_of(s * PER_TILE, 8)
        pltpu.sync_copy(tgt_hbm.at[pl.ds(base, PER_TILE)], tgt_v)
        pltpu.sync_copy(rew_hbm.at[pl.ds(base, PER_TILE)], rew_v)

        iot = lax.iota(jnp.int32, L)
        for j in range(CHUNKS):
            t = tgt_v[pl.ds(j * L, L)]
            pos = (base + j * L) + iot
            fidx_v[pl.ds(j * L, L)] = pos * V + t

        pltpu.async_copy(pred_hbm.at[fidx_v], vals_v, sem).wait()

        acc = jnp.zeros((L,), jnp.float32)
        for j in range(CHUNKS):
            acc = acc + vals_v[pl.ds(j * L, L)] * rew_v[pl.ds(j * L, L)]
        accv_v[...] = acc
        pltpu.sync_copy(accv_v, shared.at[s])

    plsc.subcore_barrier()

    @pl.when((c == 0) & (s == 0))
    def _reduce_and_write():
        pltpu.sync_copy(shared, red_v)
        tot = jnp.zeros((L,), jnp.float32)
        for r in range(NTILES):
            tot = tot + red_v[r, :]
        total = tot[0]
        for i in range(1, L):
            total = total + tot[i]
        out_v[...] = jnp.full((L,), total * (-1.0 / N), jnp.float32)
        pltpu.sync_copy(out_v, out_hbm)


@jax.jit
def _pg_loss_sc(pred1d, tgt, rew):
    mesh = plsc.VectorSubcoreMesh(core_axis_name="c", subcore_axis_name="s")
    f = functools.partial(
        pl.kernel,
        mesh=mesh,
        out_type=jax.ShapeDtypeStruct((L,), jnp.float32),
        scratch_types=[
            pltpu.VMEM((PER_TILE,), jnp.int32),
            pltpu.VMEM((PER_TILE,), jnp.float32),
            pltpu.VMEM((PER_TILE,), jnp.int32),
            pltpu.VMEM((PER_TILE,), jnp.float32),
            pltpu.VMEM((L,), jnp.float32),
            pltpu.VMEM((NTILES, L), jnp.float32),
            pltpu.VMEM((L,), jnp.float32),
            pltpu.VMEM_SHARED((NTILES, L), jnp.float32),
            pltpu.SemaphoreType.DMA,
        ],
    )(_sc_body)
    return f(pred1d, tgt, rew)


def kernel(pred, target, reward):
    pred1d = pred.reshape(-1)
    tgt = target.reshape(N).astype(jnp.int32)
    rew = reward.reshape(N).astype(jnp.float32)
    out = _pg_loss_sc(pred1d, tgt, rew)
    return out[0]


# speedup vs baseline: 23.7413x; 23.7413x over previous
import functools

import jax
import jax.numpy as jnp
from jax import lax
from jax.experimental import pallas as pl
from jax.experimental.pallas import tpu as pltpu
from jax.experimental.pallas import tpu_sc as plsc

B, S, V = 32, 32, 100000
N = B * S
L = 16
NC, NS = 2, 16
NW = NC * NS
PER_TILE = N // NW
CHUNKS = PER_TILE // L


def _sc_body(pred_hbm, tgt_hbm, rew_hbm, out_hbm,
             tgt_v, rew_v, buf_v, win_v, sem):
    c = lax.axis_index("c")
    s = lax.axis_index("s")
    wid = c * NS + s
    base = pl.multiple_of(wid * PER_TILE, 8)

    pltpu.sync_copy(tgt_hbm.at[pl.ds(base, PER_TILE)], tgt_v)
    pltpu.sync_copy(rew_hbm.at[pl.ds(base, PER_TILE)], rew_v)

    iot = lax.iota(jnp.int32, L)
    zero = jnp.zeros((L,), jnp.float32)
    copies = []
    cols = []
    lanes = []
    for j in range(CHUNKS):
        t_vec = tgt_v[pl.ds(j * L, L)]
        for i in range(L):
            k = j * L + i
            t = t_vec[i]
            v0 = pl.multiple_of(lax.bitwise_and(t, ~127), 128)
            cols.append(lax.bitwise_and(t, 127 & ~15))
            lanes.append(lax.bitwise_and(t, 15))
            copies.append(pltpu.async_copy(
                pred_hbm.at[wid, pl.ds(k & ~7, 8), pl.ds(v0, 128)],
                buf_v.at[k], sem))

    for r in range(8):
        for cchunk in range(8):
            win_v[r, pl.ds(cchunk * L, L)] = zero

    for cp in copies:
        cp.wait()

    acc = zero
    for j in range(CHUNKS):
        rew_cv = rew_v[pl.ds(j * L, L)]
        for i in range(L):
            k = j * L + i
            row16 = buf_v[k, k & 7, pl.ds(cols[k], L)]
            acc = acc + jnp.where(iot == lanes[k], rew_cv[i],
                                  jnp.float32(0.0)) * row16
    win_v[0, pl.ds(0, L)] = acc
    pltpu.sync_copy(win_v, out_hbm.at[pl.ds(wid * 8, 8)])


@jax.jit
def _pg_loss_sc(pred, tgt, rew):
    mesh = plsc.VectorSubcoreMesh(core_axis_name="c", subcore_axis_name="s")
    f = functools.partial(
        pl.kernel,
        mesh=mesh,
        out_type=jax.ShapeDtypeStruct((NW * 8, 128), jnp.float32),
        scratch_types=[
            pltpu.VMEM((PER_TILE,), jnp.int32),
            pltpu.VMEM((PER_TILE,), jnp.float32),
            pltpu.VMEM((PER_TILE, 8, 128), jnp.float32),
            pltpu.VMEM((8, 128), jnp.float32),
            pltpu.SemaphoreType.DMA,
        ],
    )(_sc_body)
    return f(pred, tgt, rew)


def _tc_reduce_body(part_ref, out_ref):
    out_ref[0, 0] = jnp.sum(part_ref[...]) * (-1.0 / N)


@jax.jit
def _tc_reduce(partials):
    return pl.pallas_call(
        _tc_reduce_body,
        out_shape=jax.ShapeDtypeStruct((1, 1), jnp.float32),
        in_specs=[pl.BlockSpec(memory_space=pltpu.VMEM)],
        out_specs=pl.BlockSpec(memory_space=pltpu.SMEM),
    )(partials)


def kernel(pred, target, reward):
    tgt = target.reshape(N).astype(jnp.int32)
    rew = reward.reshape(N).astype(jnp.float32)
    partials = _pg_loss_sc(pred, tgt, rew)
    return _tc_reduce(partials)[0, 0]
